# baseline (device time: 182679 ns/iter reference)
import jax
import jax.numpy as jnp
from jax import lax
from jax.experimental import pallas as pl
from jax.experimental.pallas import tpu as pltpu

N_DEV = 8
SQ = 256
D = 1024
DH = 128
H_LOC = 8
SCALE = 0.08838834764831843

_DID = getattr(pl, "DeviceIdType", None) or pltpu.DeviceIdType


def _attention_partial(xj, wq, wk, wv, wo):
    q = jnp.dot(xj, wq, preferred_element_type=jnp.float32).astype(jnp.bfloat16)
    k = jnp.dot(xj, wk, preferred_element_type=jnp.float32).astype(jnp.bfloat16)
    v = jnp.dot(xj, wv, preferred_element_type=jnp.float32).astype(jnp.bfloat16)
    outs = []
    for h in range(H_LOC):
        sl = slice(h * DH, (h + 1) * DH)
        qh, kh, vh = q[:, sl], k[:, sl], v[:, sl]
        s = lax.dot_general(
            qh, kh, (((1,), (1,)), ((), ())),
            preferred_element_type=jnp.float32,
        ) * SCALE
        m = jnp.max(s, axis=1, keepdims=True)
        p = jnp.exp(s - m)
        l = jnp.sum(p, axis=1, keepdims=True)
        o = jnp.dot(p.astype(jnp.bfloat16), vh,
                    preferred_element_type=jnp.float32) / l
        outs.append(o.astype(jnp.bfloat16))
    o_all = jnp.concatenate(outs, axis=1)
    return jnp.dot(o_all, wo, preferred_element_type=jnp.float32)


def kernel(x, Wq, Wo, Wk, Wv):
    x2 = x.reshape(SQ, D)

    def body(x_ref, wq_ref, wo_ref, wk_ref, wv_ref, out_ref,
             comm_ref, part_ref, rs_ref,
             ag_send, ag_recv, rs_send, rs_recv):
        my = lax.axis_index("i")
        left = (my - 1) % N_DEV
        right = (my + 1) % N_DEV

        barrier = pltpu.get_barrier_semaphore()
        for nbr in (left, right):
            pl.semaphore_signal(barrier, inc=1, device_id=(nbr,),
                                device_id_type=_DID.MESH)
        pl.semaphore_wait(barrier, 2)

        comm_ref[0] = x_ref[...].astype(jnp.bfloat16)
        for h in range(N_DEV - 1):
            rdma = pltpu.make_async_remote_copy(
                src_ref=comm_ref.at[h],
                dst_ref=comm_ref.at[h + 1],
                send_sem=ag_send.at[h],
                recv_sem=ag_recv.at[h],
                device_id=(right,),
                device_id_type=_DID.MESH,
            )
            rdma.start()
            rdma.wait()

        wq = wq_ref[...].astype(jnp.bfloat16)
        wk = wk_ref[...].astype(jnp.bfloat16)
        wv = wv_ref[...].astype(jnp.bfloat16)
        wo = wo_ref[...].astype(jnp.bfloat16)
        for h in range(N_DEV):
            part_ref[h] = _attention_partial(comm_ref[h], wq, wk, wv, wo)

        rs_ref[0] = part_ref[1]
        for t in range(N_DEV - 1):
            rdma = pltpu.make_async_remote_copy(
                src_ref=rs_ref.at[t],
                dst_ref=rs_ref.at[t + 1],
                send_sem=rs_send.at[t],
                recv_sem=rs_recv.at[t],
                device_id=(right,),
                device_id_type=_DID.MESH,
            )
            rdma.start()
            rdma.wait()
            if t < N_DEV - 2:
                rs_ref[t + 1] = rs_ref[t + 1] + part_ref[t + 2]
        out_ref[...] = rs_ref[N_DEV - 1] + part_ref[0]

    out = pl.pallas_call(
        body,
        out_shape=jax.ShapeDtypeStruct((SQ, D), jnp.float32),
        in_specs=[pl.BlockSpec(memory_space=pltpu.VMEM)] * 5,
        out_specs=pl.BlockSpec(memory_space=pltpu.VMEM),
        scratch_shapes=[
            pltpu.VMEM((N_DEV, SQ, D), jnp.bfloat16),
            pltpu.VMEM((N_DEV, SQ, D), jnp.float32),
            pltpu.VMEM((N_DEV, SQ, D), jnp.float32),
            pltpu.SemaphoreType.DMA((N_DEV,)),
            pltpu.SemaphoreType.DMA((N_DEV,)),
            pltpu.SemaphoreType.DMA((N_DEV,)),
            pltpu.SemaphoreType.DMA((N_DEV,)),
        ],
        compiler_params=pltpu.CompilerParams(collective_id=0),
    )(x2, Wq, Wo, Wk, Wv)
    return out.reshape(1, SQ, D)


# device time: 101270 ns/iter; 1.8039x vs baseline; 1.8039x over previous
import jax
import jax.numpy as jnp
from jax import lax
from jax.experimental import pallas as pl
from jax.experimental.pallas import tpu as pltpu

N_DEV = 8
SQ = 256
D = 1024
DH = 128
H_LOC = 8
SCALE = 0.08838834764831843

_DID = getattr(pl, "DeviceIdType", None) or pltpu.DeviceIdType


def _attention_partial(xj, wq, wk, wv, wo):
    q = jnp.dot(xj, wq, preferred_element_type=jnp.float32).astype(jnp.bfloat16)
    k = jnp.dot(xj, wk, preferred_element_type=jnp.float32).astype(jnp.bfloat16)
    v = jnp.dot(xj, wv, preferred_element_type=jnp.float32).astype(jnp.bfloat16)
    outs = []
    for h in range(H_LOC):
        sl = slice(h * DH, (h + 1) * DH)
        qh, kh, vh = q[:, sl], k[:, sl], v[:, sl]
        s = lax.dot_general(
            qh, kh, (((1,), (1,)), ((), ())),
            preferred_element_type=jnp.float32,
        ) * SCALE
        m = jnp.max(s, axis=1, keepdims=True)
        p = jnp.exp(s - m)
        l = jnp.sum(p, axis=1, keepdims=True)
        o = jnp.dot(p.astype(jnp.bfloat16), vh,
                    preferred_element_type=jnp.float32) / l
        outs.append(o.astype(jnp.bfloat16))
    o_all = jnp.concatenate(outs, axis=1)
    return jnp.dot(o_all, wo, preferred_element_type=jnp.float32)


def kernel(x, Wq, Wo, Wk, Wv):
    x2 = x.reshape(SQ, D)

    def body(x_ref, wq_ref, wo_ref, wk_ref, wv_ref, out_ref,
             xr_ref, xl_ref, part_ref, racc_ref, lacc_ref,
             xr_s, xr_r, xl_s, xl_r, rr_s, rr_r, ll_s, ll_r):
        my = lax.axis_index("i")
        left = (my - 1) % N_DEV
        right = (my + 1) % N_DEV

        barrier = pltpu.get_barrier_semaphore()
        for nbr in (left, right):
            pl.semaphore_signal(barrier, inc=1, device_id=(nbr,),
                                device_id_type=_DID.MESH)
        pl.semaphore_wait(barrier, 2)

        sends = []

        def rcopy(src, dst, ssem, rsem, tgt):
            d = pltpu.make_async_remote_copy(
                src_ref=src, dst_ref=dst, send_sem=ssem, recv_sem=rsem,
                device_id=(tgt,), device_id_type=_DID.MESH)
            d.start()
            sends.append(d)
            return d

        xb = x_ref[...].astype(jnp.bfloat16)
        xr_ref[0] = xb
        xl_ref[0] = xb

        gr0 = rcopy(xr_ref.at[0], xr_ref.at[1], xr_s.at[0], xr_r.at[0], right)
        gl0 = rcopy(xl_ref.at[0], xl_ref.at[1], xl_s.at[0], xl_r.at[0], left)

        wq = wq_ref[...].astype(jnp.bfloat16)
        wk = wk_ref[...].astype(jnp.bfloat16)
        wv = wv_ref[...].astype(jnp.bfloat16)
        wo = wo_ref[...].astype(jnp.bfloat16)

        def partial(xj):
            return _attention_partial(xj, wq, wk, wv, wo)

        part_ref[0] = partial(xb)

        gr0.wait_recv()
        gr1 = rcopy(xr_ref.at[1], xr_ref.at[2], xr_s.at[1], xr_r.at[1], right)
        part_ref[1] = partial(xr_ref[1])

        gl0.wait_recv()
        gl1 = rcopy(xl_ref.at[1], xl_ref.at[2], xl_s.at[1], xl_r.at[1], left)
        part_ref[7] = partial(xl_ref[1])

        gr1.wait_recv()
        gr2 = rcopy(xr_ref.at[2], xr_ref.at[3], xr_s.at[2], xr_r.at[2], right)
        part_ref[2] = partial(xr_ref[2])

        gl1.wait_recv()
        gl2 = rcopy(xl_ref.at[2], xl_ref.at[3], xl_s.at[2], xl_r.at[2], left)
        part_ref[6] = partial(xl_ref[2])

        gr2.wait_recv()
        gr3 = rcopy(xr_ref.at[3], xr_ref.at[4], xr_s.at[3], xr_r.at[3], right)
        part_ref[3] = partial(xr_ref[3])

        lacc_ref[0] = part_ref[3]
        lr0 = rcopy(lacc_ref.at[0], lacc_ref.at[1], ll_s.at[0], ll_r.at[0], left)

        gl2.wait_recv()
        part_ref[5] = partial(xl_ref[3])

        gr3.wait_recv()
        part_ref[4] = partial(xr_ref[4])

        racc_ref[0] = part_ref[4]
        rr0 = rcopy(racc_ref.at[0], racc_ref.at[1], rr_s.at[0], rr_r.at[0], right)

        lr0.wait_recv()
        lacc_ref[1] = lacc_ref[1] + part_ref[2]
        lr1 = rcopy(lacc_ref.at[1], lacc_ref.at[2], ll_s.at[1], ll_r.at[1], left)

        rr0.wait_recv()
        racc_ref[1] = racc_ref[1] + part_ref[5]
        rr1 = rcopy(racc_ref.at[1], racc_ref.at[2], rr_s.at[1], rr_r.at[1], right)

        lr1.wait_recv()
        lacc_ref[2] = lacc_ref[2] + part_ref[1]
        lr2 = rcopy(lacc_ref.at[2], lacc_ref.at[3], ll_s.at[2], ll_r.at[2], left)

        rr1.wait_recv()
        racc_ref[2] = racc_ref[2] + part_ref[6]
        rr2 = rcopy(racc_ref.at[2], racc_ref.at[3], rr_s.at[2], rr_r.at[2], right)

        rr2.wait_recv()
        racc_ref[3] = racc_ref[3] + part_ref[7]
        rr3 = rcopy(racc_ref.at[3], racc_ref.at[4], rr_s.at[3], rr_r.at[3], right)

        lr2.wait_recv()
        rr3.wait_recv()
        out_ref[...] = racc_ref[4] + lacc_ref[3] + part_ref[0]

        for d in sends:
            d.wait_send()

    out = pl.pallas_call(
        body,
        out_shape=jax.ShapeDtypeStruct((SQ, D), jnp.float32),
        in_specs=[pl.BlockSpec(memory_space=pltpu.VMEM)] * 5,
        out_specs=pl.BlockSpec(memory_space=pltpu.VMEM),
        scratch_shapes=[
            pltpu.VMEM((5, SQ, D), jnp.bfloat16),
            pltpu.VMEM((4, SQ, D), jnp.bfloat16),
            pltpu.VMEM((N_DEV, SQ, D), jnp.float32),
            pltpu.VMEM((5, SQ, D), jnp.float32),
            pltpu.VMEM((4, SQ, D), jnp.float32),
            pltpu.SemaphoreType.DMA((4,)),
            pltpu.SemaphoreType.DMA((4,)),
            pltpu.SemaphoreType.DMA((3,)),
            pltpu.SemaphoreType.DMA((3,)),
            pltpu.SemaphoreType.DMA((4,)),
            pltpu.SemaphoreType.DMA((4,)),
            pltpu.SemaphoreType.DMA((3,)),
            pltpu.SemaphoreType.DMA((3,)),
        ],
        compiler_params=pltpu.CompilerParams(collective_id=0),
    )(x2, Wq, Wo, Wk, Wv)
    return out.reshape(1, SQ, D)


# device time: 78701 ns/iter; 2.3212x vs baseline; 1.2868x over previous
import jax
import jax.numpy as jnp
from jax import lax
from jax.experimental import pallas as pl
from jax.experimental.pallas import tpu as pltpu

N_DEV = 8
SQ = 256
D = 1024
DH = 128
H_LOC = 8
SCALE = 0.08838834764831843

_DID = getattr(pl, "DeviceIdType", None) or pltpu.DeviceIdType


def _attention_partial(xj, wq, wk, wv, wo):
    q = jnp.dot(xj, wq, preferred_element_type=jnp.float32).astype(jnp.bfloat16)
    k = jnp.dot(xj, wk, preferred_element_type=jnp.float32).astype(jnp.bfloat16)
    v = jnp.dot(xj, wv, preferred_element_type=jnp.float32).astype(jnp.bfloat16)
    outs = []
    for h in range(H_LOC):
        sl = slice(h * DH, (h + 1) * DH)
        qh, kh, vh = q[:, sl], k[:, sl], v[:, sl]
        s = lax.dot_general(
            qh, kh, (((1,), (1,)), ((), ())),
            preferred_element_type=jnp.float32,
        ) * SCALE
        m = jnp.max(s, axis=1, keepdims=True)
        p = jnp.exp(s - m)
        l = jnp.sum(p, axis=1, keepdims=True)
        o = jnp.dot(p.astype(jnp.bfloat16), vh,
                    preferred_element_type=jnp.float32) / l
        outs.append(o.astype(jnp.bfloat16))
    o_all = jnp.concatenate(outs, axis=1)
    return jnp.dot(o_all, wo, preferred_element_type=jnp.float32)


def kernel(x, Wq, Wo, Wk, Wv):
    x2 = x.reshape(SQ, D)

    def body(x_ref, wq_ref, wo_ref, wk_ref, wv_ref, out_ref,
             xr_ref, xl_ref, part_ref, racc_ref, lacc_ref,
             xr_s, xr_r, xl_s, xl_r, rr_s, rr_r, ll_s, ll_r):
        my = lax.axis_index("i")
        left = (my - 1) % N_DEV
        right = (my + 1) % N_DEV

        barrier = pltpu.get_barrier_semaphore()
        for nbr in (left, right):
            pl.semaphore_signal(barrier, inc=1, device_id=(nbr,),
                                device_id_type=_DID.MESH)
        pl.semaphore_wait(barrier, 2)

        sends = []

        def rcopy(src, dst, ssem, rsem, tgt):
            d = pltpu.make_async_remote_copy(
                src_ref=src, dst_ref=dst, send_sem=ssem, recv_sem=rsem,
                device_id=(tgt,), device_id_type=_DID.MESH)
            d.start()
            sends.append(d)
            return d

        xb = x_ref[...].astype(jnp.bfloat16)
        xr_ref[0] = xb
        xl_ref[0] = xb

        gr0 = rcopy(xr_ref.at[0], xr_ref.at[1], xr_s.at[0], xr_r.at[0], right)
        gl0 = rcopy(xl_ref.at[0], xl_ref.at[1], xl_s.at[0], xl_r.at[0], left)

        wq = wq_ref[...].astype(jnp.bfloat16)
        wk = wk_ref[...].astype(jnp.bfloat16)
        wv = wv_ref[...].astype(jnp.bfloat16)
        wo = wo_ref[...].astype(jnp.bfloat16)

        def partial(xj):
            return _attention_partial(xj, wq, wk, wv, wo)

        part_ref[0] = partial(xb)

        gr0.wait_recv()
        gr1 = rcopy(xr_ref.at[1], xr_ref.at[2], xr_s.at[1], xr_r.at[1], right)
        part_ref[1] = partial(xr_ref[1])

        gl0.wait_recv()
        gl1 = rcopy(xl_ref.at[1], xl_ref.at[2], xl_s.at[1], xl_r.at[1], left)
        part_ref[7] = partial(xl_ref[1])

        gr1.wait_recv()
        gr2 = rcopy(xr_ref.at[2], xr_ref.at[3], xr_s.at[2], xr_r.at[2], right)
        part_ref[2] = partial(xr_ref[2])

        gl1.wait_recv()
        gl2 = rcopy(xl_ref.at[2], xl_ref.at[3], xl_s.at[2], xl_r.at[2], left)
        part_ref[6] = partial(xl_ref[2])

        gr2.wait_recv()
        gr3 = rcopy(xr_ref.at[3], xr_ref.at[4], xr_s.at[3], xr_r.at[3], right)
        part_ref[3] = partial(xr_ref[3])

        lacc_ref[0] = part_ref[3].astype(jnp.bfloat16)
        lr0 = rcopy(lacc_ref.at[0], lacc_ref.at[1], ll_s.at[0], ll_r.at[0], left)

        gl2.wait_recv()
        part_ref[5] = partial(xl_ref[3])

        gr3.wait_recv()
        part_ref[4] = partial(xr_ref[4])

        racc_ref[0] = part_ref[4].astype(jnp.bfloat16)
        rr0 = rcopy(racc_ref.at[0], racc_ref.at[1], rr_s.at[0], rr_r.at[0], right)

        lr0.wait_recv()
        lacc_ref[1] = (lacc_ref[1] + part_ref[2]).astype(jnp.bfloat16)
        lr1 = rcopy(lacc_ref.at[1], lacc_ref.at[2], ll_s.at[1], ll_r.at[1], left)

        rr0.wait_recv()
        racc_ref[1] = (racc_ref[1] + part_ref[5]).astype(jnp.bfloat16)
        rr1 = rcopy(racc_ref.at[1], racc_ref.at[2], rr_s.at[1], rr_r.at[1], right)

        lr1.wait_recv()
        lacc_ref[2] = (lacc_ref[2] + part_ref[1]).astype(jnp.bfloat16)
        lr2 = rcopy(lacc_ref.at[2], lacc_ref.at[3], ll_s.at[2], ll_r.at[2], left)

        rr1.wait_recv()
        racc_ref[2] = (racc_ref[2] + part_ref[6]).astype(jnp.bfloat16)
        rr2 = rcopy(racc_ref.at[2], racc_ref.at[3], rr_s.at[2], rr_r.at[2], right)

        rr2.wait_recv()
        racc_ref[3] = (racc_ref[3] + part_ref[7]).astype(jnp.bfloat16)
        rr3 = rcopy(racc_ref.at[3], racc_ref.at[4], rr_s.at[3], rr_r.at[3], right)

        lr2.wait_recv()
        rr3.wait_recv()
        out_ref[...] = (racc_ref[4].astype(jnp.float32)
                        + lacc_ref[3].astype(jnp.float32) + part_ref[0])

        for d in sends:
            d.wait_send()

    out = pl.pallas_call(
        body,
        out_shape=jax.ShapeDtypeStruct((SQ, D), jnp.float32),
        in_specs=[pl.BlockSpec(memory_space=pltpu.VMEM)] * 5,
        out_specs=pl.BlockSpec(memory_space=pltpu.VMEM),
        scratch_shapes=[
            pltpu.VMEM((5, SQ, D), jnp.bfloat16),
            pltpu.VMEM((4, SQ, D), jnp.bfloat16),
            pltpu.VMEM((N_DEV, SQ, D), jnp.float32),
            pltpu.VMEM((5, SQ, D), jnp.bfloat16),
            pltpu.VMEM((4, SQ, D), jnp.bfloat16),
            pltpu.SemaphoreType.DMA((4,)),
            pltpu.SemaphoreType.DMA((4,)),
            pltpu.SemaphoreType.DMA((3,)),
            pltpu.SemaphoreType.DMA((3,)),
            pltpu.SemaphoreType.DMA((4,)),
            pltpu.SemaphoreType.DMA((4,)),
            pltpu.SemaphoreType.DMA((3,)),
            pltpu.SemaphoreType.DMA((3,)),
        ],
        compiler_params=pltpu.CompilerParams(collective_id=0),
    )(x2, Wq, Wo, Wk, Wv)
    return out.reshape(1, SQ, D)
